# Initial kernel scaffold; baseline (speedup 1.0000x reference)
import jax
import jax.numpy as jnp
from jax.experimental import pallas as pl


def kernel(node_features, edge_index, hidden_state, W_enc, b_enc, W_l, b_l, W_r, b_r, att, gat_bias, W_ih, b_ih, W_hh, b_hh):
    raise NotImplementedError("write your pallas kernel here")


# speedup vs baseline: 568.9564x; 568.9564x over previous
import functools

import jax
import jax.numpy as jnp
from jax import lax
from jax.experimental import pallas as pl
from jax.experimental.pallas import tpu as pltpu
from jax.experimental.pallas import tpu_sc as plsc

GNN = 128
RNN = 256
HEADS = 4
CHUNK = 1000
LANES = 16
SC_CORES = 2
SC_SUBCORES = 16
SC_WORKERS = SC_CORES * SC_SUBCORES


def _sc_multiplicity(src, dst, n_pad):
    E = src.shape[0]
    epw = E // SC_WORKERS
    mesh = plsc.VectorSubcoreMesh(core_axis_name="c", subcore_axis_name="s")

    @functools.partial(
        pl.kernel,
        out_type=jax.ShapeDtypeStruct((SC_WORKERS, n_pad), jnp.float32),
        mesh=mesh,
        scratch_types=[
            pltpu.VMEM((epw,), jnp.int32),
            pltpu.VMEM((epw,), jnp.int32),
            pltpu.VMEM((n_pad,), jnp.float32),
        ],
        compiler_params=pltpu.CompilerParams(needs_layout_passes=False),
    )
    def k(src_hbm, dst_hbm, out_hbm, src_v, dst_v, m_v):
        wid = lax.axis_index("s") * SC_CORES + lax.axis_index("c")
        base = wid * epw
        pltpu.sync_copy(src_hbm.at[pl.ds(base, epw)], src_v)
        pltpu.sync_copy(dst_hbm.at[pl.ds(base, epw)], dst_v)

        zeros16 = jnp.zeros((LANES,), jnp.float32)

        def zero_body(j, carry):
            m_v[pl.ds(j * LANES, LANES)] = zeros16
            return carry

        lax.fori_loop(0, n_pad // LANES, zero_body, 0)

        ones16 = jnp.ones((LANES,), jnp.float32)

        def edge_body(i, carry):
            s16 = src_v[pl.ds(i * LANES, LANES)]
            d16 = dst_v[pl.ds(i * LANES, LANES)]
            plsc.addupdate_scatter(m_v, [s16], ones16, mask=d16 == 0)
            return carry

        lax.fori_loop(0, epw // LANES, edge_body, 0)
        pltpu.sync_copy(m_v, out_hbm.at[wid])

    return k(src, dst)


def _tc_body(nf_ref, mt_ref, hid_ref, Wenc_ref, benc_ref, Wl_ref, bl_ref,
             Wr_ref, br_ref, att_ref, gb_ref, Wih_ref, bih_ref, Whh_ref,
             bhh_ref, out_ref, alpha_s, xr0_s, amax_s, mcol_s, num_s, den_s):
    s = pl.program_id(0)
    nch = pl.num_programs(0) // 2
    n_pad = mt_ref.shape[0]

    @pl.when(s < nch)
    def _pass1():
        x = jnp.maximum(
            jnp.dot(nf_ref[...], Wenc_ref[...],
                    preferred_element_type=jnp.float32) + benc_ref[...], 0.0)
        xl = jnp.dot(x, Wl_ref[...],
                     preferred_element_type=jnp.float32) + bl_ref[...]

        @pl.when(s == 0)
        def _():
            xr0_s[...] = jnp.dot(x[0:1, :], Wr_ref[...],
                                 preferred_element_type=jnp.float32) + br_ref[...]

        xr0 = xr0_s[...]
        cols = []
        for h in range(HEADS):
            e = xl[:, h * GNN:(h + 1) * GNN] + xr0[:, h * GNN:(h + 1) * GNN]
            e = jnp.where(e >= 0.0, e, 0.2 * e)
            cols.append(jnp.sum(e * att_ref[h:h + 1, :], axis=1, keepdims=True))
        alpha_s[pl.ds(s * CHUNK, CHUNK), :] = jnp.concatenate(cols, axis=1)

    @pl.when(s == nch)
    def _mid():
        msum = jnp.sum(mt_ref[...], axis=1, keepdims=True)
        row0 = lax.broadcasted_iota(jnp.int32, (n_pad, 1), 0) == 0
        msum = msum + jnp.where(row0, 1.0, 0.0)
        mcol_s[...] = msum
        masked = jnp.where(msum > 0.0, alpha_s[...], -1e30)
        amax_s[...] = jnp.max(masked, axis=0, keepdims=True)
        num_s[...] = jnp.zeros_like(num_s)
        den_s[...] = jnp.zeros_like(den_s)

    @pl.when(s > nch)
    def _pass2():
        c = s - nch - 1
        x = jnp.maximum(
            jnp.dot(nf_ref[...], Wenc_ref[...],
                    preferred_element_type=jnp.float32) + benc_ref[...], 0.0)
        xl = jnp.dot(x, Wl_ref[...],
                     preferred_element_type=jnp.float32) + bl_ref[...]
        al = alpha_s[pl.ds(c * CHUNK, CHUNK), :]
        mc = mcol_s[pl.ds(c * CHUNK, CHUNK), :]
        w = mc * jnp.exp(jnp.minimum(al - amax_s[...], 0.0))
        den_s[...] += jnp.sum(w, axis=0, keepdims=True)
        for h in range(HEADS):
            num_s[h:h + 1, :] += jnp.sum(
                w[:, h:h + 1] * xl[:, h * GNN:(h + 1) * GNN],
                axis=0, keepdims=True)

    @pl.when(s == 2 * nch)
    def _final():
        recip = 1.0 / (den_s[...] + 1e-16)
        acc = jnp.zeros((1, GNN), jnp.float32)
        for h in range(HEADS):
            acc = acc + num_s[h:h + 1, :] * recip[0:1, h:h + 1]
        gat = acc * (1.0 / HEADS) + gb_ref[...]
        gi = jnp.dot(gat, Wih_ref[...],
                     preferred_element_type=jnp.float32) + bih_ref[...]
        gh = jnp.dot(hid_ref[...], Whh_ref[...],
                     preferred_element_type=jnp.float32) + bhh_ref[...]
        r = jax.nn.sigmoid(gi[:, 0:RNN] + gh[:, 0:RNN])
        z = jax.nn.sigmoid(gi[:, RNN:2 * RNN] + gh[:, RNN:2 * RNN])
        n = jnp.tanh(gi[:, 2 * RNN:] + r * gh[:, 2 * RNN:])
        out_ref[...] = (1.0 - z) * n + z * hid_ref[...]


def _tc_call(nf, m_t, hidden, Wenc, benc, Wl, bl, Wr, br, att, gb, Wih, bih,
             Whh, bhh, interpret=False):
    N = nf.shape[0]
    nch = N // CHUNK
    n_pad = m_t.shape[0]
    grid = 2 * nch + 1

    def chunk_of(s):
        return jnp.where(s < nch, jnp.minimum(s, nch - 1),
                         jnp.maximum(s - nch - 1, 0))

    full = lambda shp: pl.BlockSpec(shp, lambda s: tuple(0 for _ in shp))
    in_specs = [
        pl.BlockSpec((CHUNK, nf.shape[1]), lambda s: (chunk_of(s), 0)),
        full(m_t.shape),
        full(hidden.shape),
        full(Wenc.shape), full(benc.shape),
        full(Wl.shape), full(bl.shape),
        full(Wr.shape), full(br.shape),
        full(att.shape), full(gb.shape),
        full(Wih.shape), full(bih.shape),
        full(Whh.shape), full(bhh.shape),
    ]
    return pl.pallas_call(
        _tc_body,
        grid=(grid,),
        in_specs=in_specs,
        out_specs=full((1, RNN)),
        out_shape=jax.ShapeDtypeStruct((1, RNN), jnp.float32),
        scratch_shapes=[
            pltpu.VMEM((n_pad, HEADS), jnp.float32),
            pltpu.VMEM((1, HEADS * GNN), jnp.float32),
            pltpu.VMEM((1, HEADS), jnp.float32),
            pltpu.VMEM((n_pad, 1), jnp.float32),
            pltpu.VMEM((HEADS, GNN), jnp.float32),
            pltpu.VMEM((1, HEADS), jnp.float32),
        ],
        interpret=interpret,
    )(nf, m_t, hidden, Wenc, benc, Wl, bl, Wr, br, att, gb, Wih, bih, Whh, bhh)


def kernel(node_features, edge_index, hidden_state, W_enc, b_enc, W_l, b_l,
           W_r, b_r, att, gat_bias, W_ih, b_ih, W_hh, b_hh):
    N = node_features.shape[0]
    n_pad = ((N + 16 * SC_WORKERS - 1) // (16 * SC_WORKERS)) * 16 * SC_WORKERS
    ei = edge_index.astype(jnp.int32)
    m_part = _sc_multiplicity(ei[0], ei[1], n_pad)
    m_t = m_part.T
    row = lambda v: v.reshape(1, -1)
    return _tc_call(node_features, m_t, hidden_state, W_enc, row(b_enc),
                    W_l, row(b_l), W_r, row(b_r), att, row(gat_bias),
                    W_ih, row(b_ih), W_hh, row(b_hh))
